# Initial kernel scaffold; baseline (speedup 1.0000x reference)
import jax
import jax.numpy as jnp
from jax.experimental import pallas as pl


def kernel(white_idx, black_idx, white_batch, black_batch, stm, white_emb, black_emb, fc1_w, fc1_b, fc2_w, fc2_b, fc3_w, fc3_b, out_w, out_b):
    raise NotImplementedError("write your pallas kernel here")


# speedup vs baseline: 6.0679x; 6.0679x over previous
import functools

import jax
import jax.numpy as jnp
from jax import lax
from jax.experimental import pallas as pl
from jax.experimental.pallas import tpu as pltpu, tpu_sc as plsc

_N = 262144
_B = 16384
_F = 40960
_D = 256

_NC = 2
_NS = 16
_CP = 64
_NPART = _D // _CP
_CHUNK = 128
_GRP = 4
_ROWS_PT = _N // _NS // _CHUNK
_NGRP = _ROWS_PT // _GRP
_BPT = _B // _NS


def _sc_body(wi, wb, bi, bb, tw, tb, wout, bout,
             gidx, bidx, rows, acc_ref, sem0, sem1):
    c = lax.axis_index("c")
    s = lax.axis_index("s")
    sems = (sem0, sem1)

    for idx_hbm, bat_hbm, tab_hbm, out_hbm in ((wi, wb, tw, wout),
                                               (bi, bb, tb, bout)):
        pltpu.sync_copy(bat_hbm.at[pl.ds(s * _ROWS_PT, _ROWS_PT)], bidx)
        for pp in range(_NPART // _NC):
            p = c * (_NPART // _NC) + pp
            pltpu.sync_copy(idx_hbm.at[pl.ds(s * _ROWS_PT, _ROWS_PT)], gidx)

            def remap(t, _):
                r = t // 8
                col = (t % 8) * 16
                gidx[r, pl.ds(col, 16)] = gidx[r, pl.ds(col, 16)] * _NPART + p
                return 0
            lax.fori_loop(0, _ROWS_PT * 8, remap, 0)

            def zfill(t, _):
                rows[0, t // 4, pl.ds((t % 4) * 16, 16)] = jnp.zeros(
                    (16,), jnp.float32)
                return 0
            lax.fori_loop(0, _CHUNK * 4, zfill, 0)
            for k in range(_BPT // _CHUNK):
                pltpu.sync_copy(rows.at[0],
                                acc_ref.at[pl.ds(s * _BPT + k * _CHUNK,
                                                 _CHUNK)])
            plsc.subcore_barrier()

            pltpu.async_copy(tab_hbm.at[gidx.at[0]], rows.at[0], sems[0])

            def pair(it, _):
                c0 = it * 2
                for k in range(2):
                    ch = c0 + k
                    par = k
                    npar = 1 - k

                    @pl.when(ch + 1 < _ROWS_PT)
                    def _fire():
                        cn = jnp.minimum(ch + 1, _ROWS_PT - 1)
                        pltpu.async_copy(tab_hbm.at[gidx.at[cn]],
                                         rows.at[npar], sems[npar])
                    pltpu.make_async_copy(tab_hbm.at[gidx.at[ch]],
                                          rows.at[par], sems[par]).wait()
                    pltpu.sync_copy(rows.at[par], acc_ref.at[bidx.at[ch]],
                                    add=True)
                return 0
            lax.fori_loop(0, _ROWS_PT // 2, pair, 0)
            plsc.subcore_barrier()

            for k in range(_BPT // _CHUNK):
                r0 = s * _BPT + k * _CHUNK
                pltpu.sync_copy(acc_ref.at[pl.ds(r0, _CHUNK)], rows.at[0])
                pltpu.sync_copy(rows.at[0], out_hbm.at[p, pl.ds(r0, _CHUNK)])
            plsc.subcore_barrier()


@functools.lru_cache(maxsize=None)
def _sc_segsum():
    return pl.kernel(
        _sc_body,
        out_type=(jax.ShapeDtypeStruct((_NPART, _B, _CP), jnp.float32),
                  jax.ShapeDtypeStruct((_NPART, _B, _CP), jnp.float32)),
        mesh=plsc.VectorSubcoreMesh(core_axis_name="c", subcore_axis_name="s",
                                    num_cores=_NC, num_subcores=_NS),
        scratch_types=(
            pltpu.VMEM((_ROWS_PT, _CHUNK), jnp.int32),
            pltpu.VMEM((_ROWS_PT, _CHUNK), jnp.int32),
            pltpu.VMEM((2, _CHUNK, _CP), jnp.float32),
            pltpu.VMEM_SHARED((_B, _CP), jnp.float32),
            pltpu.SemaphoreType.DMA,
            pltpu.SemaphoreType.DMA,
        ),
        compiler_params=pltpu.CompilerParams(use_tc_tiling_on_sc=False),
    )


_BLK = 2048


def _mlp_body(wp, bp, stm_r, w1f, w1s, b1, w2, b2, w3, b3, w4, b4, out_r):
    w = jnp.concatenate([wp[q] for q in range(_NPART)], axis=-1)
    b = jnp.concatenate([bp[q] for q in range(_NPART)], axis=-1)
    stm1 = stm_r[...] > 0
    first = jnp.where(stm1, w, b)
    second = jnp.where(stm1, b, w)
    x = jnp.maximum(first @ w1f[...] + second @ w1s[...] + b1[...], 0.0)
    x = jnp.maximum(x @ w2[...] + b2[...], 0.0)
    x = jnp.maximum(x @ w3[...] + b3[...], 0.0)
    out_r[...] = jnp.sum(x * w4[...], axis=1, keepdims=True) + b4[...]


def _mlp(wparts, bparts, stm2, w1f, w1s, b1, w2, b2, w3, b3, w4, b4):
    rep = lambda i: (0, 0)
    return pl.pallas_call(
        _mlp_body,
        grid=(_B // _BLK,),
        in_specs=[
            pl.BlockSpec((_NPART, _BLK, _CP), lambda i: (0, i, 0)),
            pl.BlockSpec((_NPART, _BLK, _CP), lambda i: (0, i, 0)),
            pl.BlockSpec((_BLK, 1), lambda i: (i, 0)),
            pl.BlockSpec((_D, 128), rep),
            pl.BlockSpec((_D, 128), rep),
            pl.BlockSpec((1, 128), rep),
            pl.BlockSpec((128, 64), rep),
            pl.BlockSpec((1, 64), rep),
            pl.BlockSpec((64, 32), rep),
            pl.BlockSpec((1, 32), rep),
            pl.BlockSpec((1, 32), rep),
            pl.BlockSpec((1, 1), rep),
        ],
        out_specs=pl.BlockSpec((_BLK, 1), lambda i: (i, 0)),
        out_shape=jax.ShapeDtypeStruct((_B, 1), jnp.float32),
    )(wparts, bparts, stm2, w1f, w1s, b1, w2, b2, w3, b3, w4, b4)


def kernel(white_idx, black_idx, white_batch, black_batch, stm,
           white_emb, black_emb, fc1_w, fc1_b, fc2_w, fc2_b,
           fc3_w, fc3_b, out_w, out_b):
    wi = white_idx.reshape(-1, _CHUNK).astype(jnp.int32)
    bi = black_idx.reshape(-1, _CHUNK).astype(jnp.int32)
    wb = white_batch.reshape(-1, _CHUNK).astype(jnp.int32)
    bb = black_batch.reshape(-1, _CHUNK).astype(jnp.int32)
    tw = white_emb.reshape(_F * _NPART, _CP)
    tb = black_emb.reshape(_F * _NPART, _CP)

    wparts, bparts = _sc_segsum()(wi, wb, bi, bb, tw, tb)

    w1t = fc1_w.T
    return _mlp(wparts, bparts, stm.reshape(_B, 1).astype(jnp.int32),
                w1t[:_D], w1t[_D:], fc1_b.reshape(1, -1),
                fc2_w.T, fc2_b.reshape(1, -1),
                fc3_w.T, fc3_b.reshape(1, -1),
                out_w, out_b.reshape(1, 1))
